# initial kernel scaffold (unmeasured)
import jax
import jax.numpy as jnp
from jax import lax
from jax.experimental import pallas as pl
from jax.experimental.pallas import tpu as pltpu


def kernel(
    x,
):
    def body(*refs):
        pass

    out_shape = jax.ShapeDtypeStruct(..., jnp.float32)
    return pl.pallas_call(body, out_shape=out_shape)(...)


# baseline (device time: 79539 ns/iter reference)
import jax
import jax.numpy as jnp
from jax import lax
from jax.experimental import pallas as pl
from jax.experimental.pallas import tpu as pltpu

M = 1024
N = 1024
RS_ROWS = (512, 256, 128, 64, 32)


def _partners_and_bits(i):
    z = i // 8
    p = i % 8
    y = p // 2
    x = (p + y) % 2

    def logical(xx, yy, zz):
        return zz * 8 + 2 * yy + (xx + yy) % 2

    phases = [
        (logical(1 - x, y, z), x),
        (logical(x, y ^ 1, z), y & 1),
        (logical(x, y, z ^ 1), z & 1),
        (logical(x, y ^ 2, z), (y >> 1) & 1),
        (logical(x, y, z ^ 2), (z >> 1) & 1),
    ]
    return phases


def kernel(x):
    def body(x_ref, out_ref, *scratch):
        send_bufs = scratch[0:5]
        recv_bufs = scratch[5:10]
        ag_send_bufs = scratch[10:15]
        ag_recv_bufs = scratch[15:20]
        send_sems, recv_sems = scratch[20], scratch[21]

        i = lax.axis_index("i")
        phases = _partners_and_bits(i)

        out_ref[:, :] = x_ref[0, :, :]

        off = jnp.int32(0)
        for k, (partner, bit) in enumerate(phases):
            n = RS_ROWS[k]
            keep_off = off + bit * n
            send_off = off + (1 - bit) * n
            send_bufs[k][:, :] = out_ref[pl.ds(send_off, n), :].astype(
                jnp.bfloat16
            )
            rdma = pltpu.make_async_remote_copy(
                src_ref=send_bufs[k],
                dst_ref=recv_bufs[k],
                send_sem=send_sems.at[k],
                recv_sem=recv_sems.at[k],
                device_id=(partner,),
                device_id_type=pl.DeviceIdType.MESH,
            )
            rdma.start()
            rdma.wait()
            out_ref[pl.ds(keep_off, n), :] = out_ref[
                pl.ds(keep_off, n), :
            ] + recv_bufs[k][:, :].astype(jnp.float32)
            off = keep_off

        for k in reversed(range(5)):
            partner, bit = phases[k]
            n = RS_ROWS[k]
            partner_off = off + (1 - 2 * bit) * n
            ag_send_bufs[k][:, :] = out_ref[pl.ds(off, n), :].astype(
                jnp.bfloat16
            )
            rdma = pltpu.make_async_remote_copy(
                src_ref=ag_send_bufs[k],
                dst_ref=ag_recv_bufs[k],
                send_sem=send_sems.at[5 + k],
                recv_sem=recv_sems.at[5 + k],
                device_id=(partner,),
                device_id_type=pl.DeviceIdType.MESH,
            )
            rdma.start()
            rdma.wait()
            out_ref[pl.ds(partner_off, n), :] = ag_recv_bufs[k][:, :].astype(
                jnp.float32
            )
            off = off - bit * n

    comm = [pltpu.VMEM((n, N), jnp.bfloat16) for n in RS_ROWS]
    return pl.pallas_call(
        body,
        out_shape=jax.ShapeDtypeStruct((M, N), jnp.float32),
        in_specs=[pl.BlockSpec(memory_space=pltpu.VMEM)],
        out_specs=pl.BlockSpec(memory_space=pltpu.VMEM),
        scratch_shapes=(
            comm
            + comm
            + comm
            + comm
            + [
                pltpu.SemaphoreType.DMA((10,)),
                pltpu.SemaphoreType.DMA((10,)),
            ]
        ),
    )(x)


# device time: 59113 ns/iter; 1.3455x vs baseline; 1.3455x over previous
import jax
import jax.numpy as jnp
from jax import lax
from jax.experimental import pallas as pl
from jax.experimental.pallas import tpu as pltpu

M = 1024
N = 1024
HALF = 512
ROWS = (256, 128, 64, 32, 16)
ORDERS = (("x", "y1", "z1", "y2", "z2"), ("y1", "z1", "x", "z2", "y2"))


def _phases(i):
    z = i // 8
    p = i % 8
    y = p // 2
    x = (p + y) % 2

    def logical(xx, yy, zz):
        return zz * 8 + 2 * yy + (xx + yy) % 2

    return {
        "x": (logical(1 - x, y, z), x),
        "y1": (logical(x, y ^ 1, z), y & 1),
        "z1": (logical(x, y, z ^ 1), z & 1),
        "y2": (logical(x, y ^ 2, z), (y >> 1) & 1),
        "z2": (logical(x, y, z ^ 2), (z >> 1) & 1),
    }


def kernel(x):
    def body(x_ref, out_ref, *scratch):
        def group(g):
            return (scratch[g * 10 : g * 10 + 5], scratch[g * 10 + 5 : g * 10 + 10])

        rs_send = group(0)
        rs_recv = group(1)
        ag_send = group(2)
        ag_recv = group(3)
        send_sems, recv_sems = scratch[40], scratch[41]

        i = lax.axis_index("i")
        dims = _phases(i)

        out_ref[:, :] = x_ref[0, :, :]

        off = [jnp.int32(0), jnp.int32(HALF)]

        for s in range(5):
            n = ROWS[s]
            rdmas = []
            keeps = []
            for h in (0, 1):
                partner, bit = dims[ORDERS[h][s]]
                keep_off = pl.multiple_of(off[h] + bit * n, 16)
                send_off = pl.multiple_of(off[h] + (1 - bit) * n, 16)
                rs_send[h][s][:, :] = out_ref[pl.ds(send_off, n), :].astype(
                    jnp.bfloat16
                )
                rdma = pltpu.make_async_remote_copy(
                    src_ref=rs_send[h][s],
                    dst_ref=rs_recv[h][s],
                    send_sem=send_sems.at[h * 5 + s],
                    recv_sem=recv_sems.at[h * 5 + s],
                    device_id=(partner,),
                    device_id_type=pl.DeviceIdType.MESH,
                )
                rdma.start()
                rdmas.append(rdma)
                keeps.append(keep_off)
                off[h] = keep_off
            for h in (0, 1):
                rdmas[h].wait()
                out_ref[pl.ds(keeps[h], n), :] = out_ref[
                    pl.ds(keeps[h], n), :
                ] + rs_recv[h][s][:, :].astype(jnp.float32)

        for s in reversed(range(5)):
            n = ROWS[s]
            rdmas = []
            partner_offs = []
            for h in (0, 1):
                partner, bit = dims[ORDERS[h][s]]
                partner_off = pl.multiple_of(off[h] + (1 - 2 * bit) * n, 16)
                my_off = pl.multiple_of(off[h], 16)
                ag_send[h][s][:, :] = out_ref[pl.ds(my_off, n), :].astype(
                    jnp.bfloat16
                )
                rdma = pltpu.make_async_remote_copy(
                    src_ref=ag_send[h][s],
                    dst_ref=ag_recv[h][s],
                    send_sem=send_sems.at[10 + h * 5 + s],
                    recv_sem=recv_sems.at[10 + h * 5 + s],
                    device_id=(partner,),
                    device_id_type=pl.DeviceIdType.MESH,
                )
                rdma.start()
                rdmas.append(rdma)
                partner_offs.append(partner_off)
                off[h] = off[h] - bit * n
            for h in (0, 1):
                rdmas[h].wait()
                out_ref[pl.ds(partner_offs[h], n), :] = ag_recv[h][s][
                    :, :
                ].astype(jnp.float32)

    comm = [
        pltpu.VMEM((n, N), jnp.bfloat16) for _ in (0, 1) for n in ROWS
    ]
    return pl.pallas_call(
        body,
        out_shape=jax.ShapeDtypeStruct((M, N), jnp.float32),
        in_specs=[pl.BlockSpec(memory_space=pltpu.VMEM)],
        out_specs=pl.BlockSpec(memory_space=pltpu.VMEM),
        scratch_shapes=(
            comm
            + comm
            + comm
            + comm
            + [
                pltpu.SemaphoreType.DMA((20,)),
                pltpu.SemaphoreType.DMA((20,)),
            ]
        ),
    )(x)


# device time: 52762 ns/iter; 1.5075x vs baseline; 1.1204x over previous
import jax
import jax.numpy as jnp
from jax import lax
from jax.experimental import pallas as pl
from jax.experimental.pallas import tpu as pltpu

M = 1024
N = 1024
HALF = 512
ROWS = (256, 128, 64, 32, 16)
ORDERS = (("x", "y1", "z1", "y2", "z2"), ("y1", "z1", "x", "z2", "y2"))


def _phases(i):
    z = i // 8
    p = i % 8
    y = p // 2
    x = (p + y) % 2

    def logical(xx, yy, zz):
        return zz * 8 + 2 * yy + (xx + yy) % 2

    return {
        "x": (logical(1 - x, y, z), x),
        "y1": (logical(x, y ^ 1, z), y & 1),
        "z1": (logical(x, y, z ^ 1), z & 1),
        "y2": (logical(x, y ^ 2, z), (y >> 1) & 1),
        "z2": (logical(x, y, z ^ 2), (z >> 1) & 1),
    }


def kernel(x):
    def body(x_ref, out_ref, *scratch):
        def group(g):
            return (scratch[g * 10 : g * 10 + 5], scratch[g * 10 + 5 : g * 10 + 10])

        rs_send = group(0)
        rs_recv = group(1)
        ag_send = group(2)
        ag_recv = group(3)
        send_sems, recv_sems = scratch[40], scratch[41]

        i = lax.axis_index("i")
        dims = _phases(i)

        barrier_sem = pltpu.get_barrier_semaphore()
        for d in ("x", "y1", "z1", "y2", "z2"):
            pl.semaphore_signal(
                barrier_sem,
                inc=1,
                device_id=(dims[d][0],),
                device_id_type=pl.DeviceIdType.MESH,
            )
        pl.semaphore_wait(barrier_sem, 5)

        off = [jnp.int32(0), jnp.int32(HALF)]

        for s in range(5):
            n = ROWS[s]
            rdmas = []
            keeps = []
            for h in (0, 1):
                partner, bit = dims[ORDERS[h][s]]
                keep_off = pl.multiple_of(off[h] + bit * n, 16)
                send_off = pl.multiple_of(off[h] + (1 - bit) * n, 16)
                src = x_ref.at[0] if s == 0 else out_ref
                rs_send[h][s][:, :] = src[pl.ds(send_off, n), :].astype(
                    jnp.bfloat16
                )
                rdma = pltpu.make_async_remote_copy(
                    src_ref=rs_send[h][s],
                    dst_ref=rs_recv[h][s],
                    send_sem=send_sems.at[h * 5 + s],
                    recv_sem=recv_sems.at[h * 5 + s],
                    device_id=(partner,),
                    device_id_type=pl.DeviceIdType.MESH,
                )
                rdma.start()
                rdmas.append(rdma)
                keeps.append(keep_off)
                off[h] = keep_off
            for h in (0, 1):
                rdmas[h].wait()
                base = x_ref.at[0] if s == 0 else out_ref
                out_ref[pl.ds(keeps[h], n), :] = base[
                    pl.ds(keeps[h], n), :
                ] + rs_recv[h][s][:, :].astype(jnp.float32)

        for s in reversed(range(5)):
            n = ROWS[s]
            rdmas = []
            partner_offs = []
            for h in (0, 1):
                partner, bit = dims[ORDERS[h][s]]
                partner_off = pl.multiple_of(off[h] + (1 - 2 * bit) * n, 16)
                my_off = pl.multiple_of(off[h], 16)
                ag_send[h][s][:, :] = out_ref[pl.ds(my_off, n), :].astype(
                    jnp.bfloat16
                )
                rdma = pltpu.make_async_remote_copy(
                    src_ref=ag_send[h][s],
                    dst_ref=ag_recv[h][s],
                    send_sem=send_sems.at[10 + h * 5 + s],
                    recv_sem=recv_sems.at[10 + h * 5 + s],
                    device_id=(partner,),
                    device_id_type=pl.DeviceIdType.MESH,
                )
                rdma.start()
                rdmas.append(rdma)
                partner_offs.append(partner_off)
                off[h] = off[h] - bit * n
            for h in (0, 1):
                rdmas[h].wait()
                out_ref[pl.ds(partner_offs[h], n), :] = ag_recv[h][s][
                    :, :
                ].astype(jnp.float32)

    comm = [
        pltpu.VMEM((n, N), jnp.bfloat16) for _ in (0, 1) for n in ROWS
    ]
    return pl.pallas_call(
        body,
        out_shape=jax.ShapeDtypeStruct((M, N), jnp.float32),
        in_specs=[pl.BlockSpec(memory_space=pltpu.VMEM)],
        out_specs=pl.BlockSpec(memory_space=pltpu.VMEM),
        scratch_shapes=(
            comm
            + comm
            + comm
            + comm
            + [
                pltpu.SemaphoreType.DMA((20,)),
                pltpu.SemaphoreType.DMA((20,)),
            ]
        ),
        compiler_params=pltpu.CompilerParams(collective_id=0),
    )(x)


# device time: 51865 ns/iter; 1.5336x vs baseline; 1.0173x over previous
import jax
import jax.numpy as jnp
from jax import lax
from jax.experimental import pallas as pl
from jax.experimental.pallas import tpu as pltpu

M = 1024
N = 1024
HALF = 512
ROWS = (256, 128, 64, 32, 16)
ORDERS = (("x", "y1", "z1", "y2", "z2"), ("y1", "z1", "x", "z2", "y2"))


def _phases(i):
    z = i // 8
    p = i % 8
    y = p // 2
    x = (p + y) % 2

    def logical(xx, yy, zz):
        return zz * 8 + 2 * yy + (xx + yy) % 2

    return {
        "x": (logical(1 - x, y, z), x),
        "y1": (logical(x, y ^ 1, z), y & 1),
        "z1": (logical(x, y, z ^ 1), z & 1),
        "y2": (logical(x, y ^ 2, z), (y >> 1) & 1),
        "z2": (logical(x, y, z ^ 2), (z >> 1) & 1),
    }


def kernel(x):
    def body(x_ref, out_ref, *scratch):
        def group(g):
            return (scratch[g * 10 : g * 10 + 5], scratch[g * 10 + 5 : g * 10 + 10])

        rs_send = group(0)
        rs_recv = group(1)
        ag_send = group(2)
        ag_recv = group(3)
        send_sems, recv_sems = scratch[40], scratch[41]

        i = lax.axis_index("i")
        dims = _phases(i)

        barrier_sem = pltpu.get_barrier_semaphore()
        for d in ("x", "y1", "z1", "y2", "z2"):
            pl.semaphore_signal(
                barrier_sem,
                inc=1,
                device_id=(dims[d][0],),
                device_id_type=pl.DeviceIdType.MESH,
            )
        pl.semaphore_wait(barrier_sem, 5)

        off = [jnp.int32(0), jnp.int32(HALF)]

        for s in range(5):
            n = ROWS[s]
            rdmas = []
            keeps = []
            for h in (0, 1):
                partner, bit = dims[ORDERS[h][s]]
                keep_off = pl.multiple_of(off[h] + bit * n, 16)
                send_off = pl.multiple_of(off[h] + (1 - bit) * n, 16)
                rdma = pltpu.make_async_remote_copy(
                    src_ref=rs_send[h][s],
                    dst_ref=rs_recv[h][s],
                    send_sem=send_sems.at[h * 5 + s],
                    recv_sem=recv_sems.at[h * 5 + s],
                    device_id=(partner,),
                    device_id_type=pl.DeviceIdType.MESH,
                )
                rdma.start()
                rdmas.append(rdma)
                keeps.append(keep_off)
                off[h] = keep_off
            for h in (0, 1):
                rdmas[h].wait()

        for s in reversed(range(5)):
            n = ROWS[s]
            rdmas = []
            partner_offs = []
            for h in (0, 1):
                partner, bit = dims[ORDERS[h][s]]
                partner_off = pl.multiple_of(off[h] + (1 - 2 * bit) * n, 16)
                rdma = pltpu.make_async_remote_copy(
                    src_ref=ag_send[h][s],
                    dst_ref=ag_recv[h][s],
                    send_sem=send_sems.at[10 + h * 5 + s],
                    recv_sem=recv_sems.at[10 + h * 5 + s],
                    device_id=(partner,),
                    device_id_type=pl.DeviceIdType.MESH,
                )
                rdma.start()
                rdmas.append(rdma)
                partner_offs.append(partner_off)
                off[h] = off[h] - bit * n
            for h in (0, 1):
                rdmas[h].wait()

    comm = [
        pltpu.VMEM((n, N), jnp.bfloat16) for _ in (0, 1) for n in ROWS
    ]
    return pl.pallas_call(
        body,
        out_shape=jax.ShapeDtypeStruct((M, N), jnp.float32),
        in_specs=[pl.BlockSpec(memory_space=pltpu.VMEM)],
        out_specs=pl.BlockSpec(memory_space=pltpu.VMEM),
        scratch_shapes=(
            comm
            + comm
            + comm
            + comm
            + [
                pltpu.SemaphoreType.DMA((20,)),
                pltpu.SemaphoreType.DMA((20,)),
            ]
        ),
        compiler_params=pltpu.CompilerParams(collective_id=0),
    )(x)


# device time: 44302 ns/iter; 1.7954x vs baseline; 1.1707x over previous
import jax
import jax.numpy as jnp
from jax import lax
from jax.experimental import pallas as pl
from jax.experimental.pallas import tpu as pltpu

M = 1024
N = 1024
HALF = 512
ROWS = (256, 128, 64, 32, 16)
ORDERS = (("x", "y1", "z1", "y2", "z2"), ("y1", "z1", "x", "z2", "y2"))

_SHAPES: list[int] = []
_IDX: dict[str, int] = {}


def _buf(name: str, rows: int) -> None:
    _IDX[name] = len(_SHAPES)
    _SHAPES.append(rows)


for _h in (0, 1):
    for _s in range(4):
        _m = ROWS[_s + 1]
        for _tag in ("rs_sf", "rs_sr", "rs_rf", "rs_rr"):
            _buf(f"{_tag}{_h}{_s}", _m)
    _buf(f"rs_s4_{_h}", 16)
    _buf(f"rs_r4_{_h}", 16)
    _buf(f"ag_s4_{_h}", 16)
    _buf(f"ag_r4_{_h}", 16)
    for _k in range(4):
        _m = ROWS[_k + 1]
        for _tag in ("ag_so", "ag_sn", "ag_ro", "ag_rn"):
            _buf(f"{_tag}{_h}{_k}", _m)

NSEM = 36


def _phases(i):
    z = i // 8
    p = i % 8
    y = p // 2
    x = (p + y) % 2

    def logical(xx, yy, zz):
        return zz * 8 + 2 * yy + (xx + yy) % 2

    return {
        "x": (logical(1 - x, y, z), x),
        "y1": (logical(x, y ^ 1, z), y & 1),
        "z1": (logical(x, y, z ^ 1), z & 1),
        "y2": (logical(x, y ^ 2, z), (y >> 1) & 1),
        "z2": (logical(x, y, z ^ 2), (z >> 1) & 1),
    }


def kernel(x):
    def body(x_ref, out_ref, *scratch):
        bufs, send_sems, recv_sems = scratch[:-2], scratch[-2], scratch[-1]

        def B(name):
            return bufs[_IDX[name]]

        sem_ctr = [0]

        def rdma(src_name, dst_name, partner):
            j = sem_ctr[0]
            sem_ctr[0] += 1
            r = pltpu.make_async_remote_copy(
                src_ref=B(src_name),
                dst_ref=B(dst_name),
                send_sem=send_sems.at[j],
                recv_sem=recv_sems.at[j],
                device_id=(partner,),
                device_id_type=pl.DeviceIdType.MESH,
            )
            r.start()
            return r

        def stage(name, src, off, m):
            B(name)[:, :] = src[pl.ds(pl.multiple_of(off, 16), m), :].astype(
                jnp.bfloat16
            )

        def addin(off, m, name, base):
            off = pl.multiple_of(off, 16)
            out_ref[pl.ds(off, m), :] = base[pl.ds(off, m), :] + B(name)[
                :, :
            ].astype(jnp.float32)

        def store(off, m, name):
            out_ref[pl.ds(pl.multiple_of(off, 16), m), :] = B(name)[
                :, :
            ].astype(jnp.float32)

        i = lax.axis_index("i")
        dims = _phases(i)
        x0 = x_ref.at[0]

        barrier_sem = pltpu.get_barrier_semaphore()
        for d in ("x", "y1", "z1", "y2", "z2"):
            pl.semaphore_signal(
                barrier_sem,
                inc=1,
                device_id=(dims[d][0],),
                device_id_type=pl.DeviceIdType.MESH,
            )
        pl.semaphore_wait(barrier_sem, 5)

        P = []
        for h in (0, 1):
            bit = [dims[ORDERS[h][s]][1] for s in range(5)]
            par = [dims[ORDERS[h][s]][0] for s in range(5)]
            off = jnp.int32(HALF * h)
            keep, send = [], []
            for s in range(5):
                keep.append(off + bit[s] * ROWS[s])
                send.append(off + (1 - bit[s]) * ROWS[s])
                off = keep[s]
            sf, sr, kf, kr = [], [], [], []
            for s in range(4):
                m, bp = ROWS[s + 1], bit[s + 1]
                sf.append(send[s] + (1 - bp) * m)
                sr.append(send[s] + bp * m)
                kf.append(keep[s] + (1 - bp) * m)
                kr.append(keep[s] + bp * m)
            o, po = [None] * 5, [None] * 5
            o[4] = keep[4]
            for k in range(4, -1, -1):
                po[k] = o[k] + (1 - 2 * bit[k]) * ROWS[k]
                if k:
                    o[k - 1] = o[k] - bit[k] * ROWS[k]
            P.append(
                dict(bit=bit, par=par, keep=keep, send=send, sf=sf, sr=sr,
                     kf=kf, kr=kr, o=o, po=po)
            )

        rs_f = [[None] * 4, [None] * 4]
        rs_r = [[None] * 4, [None] * 4]
        rs4 = [None, None]
        for h in (0, 1):
            stage(f"rs_sf{h}0", x0, P[h]["sf"][0], ROWS[1])
            rs_f[h][0] = rdma(f"rs_sf{h}0", f"rs_rf{h}0", P[h]["par"][0])
        for h in (0, 1):
            stage(f"rs_sr{h}0", x0, P[h]["sr"][0], ROWS[1])
            rs_r[h][0] = rdma(f"rs_sr{h}0", f"rs_rr{h}0", P[h]["par"][0])
        for s in range(4):
            m = ROWS[s + 1]
            for h in (0, 1):
                rs_f[h][s].wait()
                addin(P[h]["kf"][s], m, f"rs_rf{h}{s}",
                      x0 if s == 0 else out_ref)
            for h in (0, 1):
                if s < 3:
                    m2 = ROWS[s + 2]
                    stage(f"rs_sf{h}{s + 1}", out_ref, P[h]["sf"][s + 1], m2)
                    rs_f[h][s + 1] = rdma(
                        f"rs_sf{h}{s + 1}", f"rs_rf{h}{s + 1}",
                        P[h]["par"][s + 1],
                    )
                    stage(f"rs_sr{h}{s + 1}", out_ref, P[h]["sr"][s + 1], m2)
                    rs_r[h][s + 1] = rdma(
                        f"rs_sr{h}{s + 1}", f"rs_rr{h}{s + 1}",
                        P[h]["par"][s + 1],
                    )
                else:
                    stage(f"rs_s4_{h}", out_ref, P[h]["send"][4], 16)
                    rs4[h] = rdma(f"rs_s4_{h}", f"rs_r4_{h}", P[h]["par"][4])
            for h in (0, 1):
                rs_r[h][s].wait()
                addin(P[h]["kr"][s], m, f"rs_rr{h}{s}",
                      x0 if s == 0 else out_ref)
        for h in (0, 1):
            rs4[h].wait()
            addin(P[h]["keep"][4], 16, f"rs_r4_{h}", out_ref)

        ag4 = [None, None]
        agO = [[None] * 4, [None] * 4]
        agN = [[None] * 4, [None] * 4]
        for h in (0, 1):
            stage(f"ag_s4_{h}", out_ref, P[h]["o"][4], 16)
            ag4[h] = rdma(f"ag_s4_{h}", f"ag_r4_{h}", P[h]["par"][4])
            stage(f"ag_so{h}3", out_ref, P[h]["o"][4], 16)
            agO[h][3] = rdma(f"ag_so{h}3", f"ag_ro{h}3", P[h]["par"][3])
        for k in range(3, -1, -1):
            m = ROWS[k + 1]
            if k == 3:
                for h in (0, 1):
                    ag4[h].wait()
                    store(P[h]["po"][4], 16, f"ag_r4_{h}")
            else:
                m2 = ROWS[k + 2]
                for h in (0, 1):
                    b = P[h]["bit"][k + 2]
                    agO[h][k + 1].wait()
                    store(P[h]["po"][k + 1] + b * m2, m2, f"ag_ro{h}{k + 1}")
                    agN[h][k + 1].wait()
                    store(P[h]["po"][k + 1] + (1 - b) * m2, m2,
                          f"ag_rn{h}{k + 1}")
            for h in (0, 1):
                stage(f"ag_sn{h}{k}", out_ref, P[h]["po"][k + 1], m)
                agN[h][k] = rdma(f"ag_sn{h}{k}", f"ag_rn{h}{k}",
                                 P[h]["par"][k])
                if k >= 1:
                    stage(f"ag_so{h}{k - 1}", out_ref, P[h]["o"][k], ROWS[k])
                    agO[h][k - 1] = rdma(
                        f"ag_so{h}{k - 1}", f"ag_ro{h}{k - 1}",
                        P[h]["par"][k - 1],
                    )
        for h in (0, 1):
            b = P[h]["bit"][1]
            m2 = ROWS[1]
            agO[h][0].wait()
            store(P[h]["po"][0] + b * m2, m2, f"ag_ro{h}0")
            agN[h][0].wait()
            store(P[h]["po"][0] + (1 - b) * m2, m2, f"ag_rn{h}0")

    return pl.pallas_call(
        body,
        out_shape=jax.ShapeDtypeStruct((M, N), jnp.float32),
        in_specs=[pl.BlockSpec(memory_space=pltpu.VMEM)],
        out_specs=pl.BlockSpec(memory_space=pltpu.VMEM),
        scratch_shapes=(
            [pltpu.VMEM((r, N), jnp.bfloat16) for r in _SHAPES]
            + [
                pltpu.SemaphoreType.DMA((NSEM,)),
                pltpu.SemaphoreType.DMA((NSEM,)),
            ]
        ),
        compiler_params=pltpu.CompilerParams(collective_id=0),
    )(x)


# device time: 42232 ns/iter; 1.8834x vs baseline; 1.0490x over previous
import jax
import jax.numpy as jnp
from jax import lax
from jax.experimental import pallas as pl
from jax.experimental.pallas import tpu as pltpu

M = 1024
N = 1024
HALF = 512
ROWS = (256, 128, 64, 32, 16)
ORDERS = (("x", "y1", "z1", "y2", "z2"), ("y1", "z1", "x", "z2", "y2"))

_SHAPES: list[int] = []
_IDX: dict[str, int] = {}


def _buf(name: str, rows: int) -> None:
    _IDX[name] = len(_SHAPES)
    _SHAPES.append(rows)


for _h in (0, 1):
    for _s in range(4):
        _m = ROWS[_s + 1]
        for _tag in ("rs_sf", "rs_sr", "rs_rf", "rs_rr"):
            _buf(f"{_tag}{_h}{_s}", _m)
    for _tag in ("m4_sa", "m4_sb", "m4_ra", "m4_rb",
                 "ag3_sa", "ag3_sb", "ag3_ra", "ag3_rb"):
        _buf(f"{_tag}_{_h}", 16)
    for _k in range(3):
        _m = ROWS[_k + 1]
        for _tag in ("ag_so", "ag_sn", "ag_ro", "ag_rn"):
            _buf(f"{_tag}{_h}{_k}", _m)

NSEM = 36


def _phases(i):
    z = i // 8
    p = i % 8
    y = p // 2
    x = (p + y) % 2

    def logical(xx, yy, zz):
        return zz * 8 + 2 * yy + (xx + yy) % 2

    return {
        "x": (logical(1 - x, y, z), x),
        "y1": (logical(x, y ^ 1, z), y & 1),
        "z1": (logical(x, y, z ^ 1), z & 1),
        "y2": (logical(x, y ^ 2, z), (y >> 1) & 1),
        "z2": (logical(x, y, z ^ 2), (z >> 1) & 1),
    }


def kernel(x):
    def body(x_ref, out_ref, *scratch):
        bufs, send_sems, recv_sems = scratch[:-2], scratch[-2], scratch[-1]

        def B(name):
            return bufs[_IDX[name]]

        sem_ctr = [0]

        def rdma(src_name, dst_name, partner):
            j = sem_ctr[0]
            sem_ctr[0] += 1
            r = pltpu.make_async_remote_copy(
                src_ref=B(src_name),
                dst_ref=B(dst_name),
                send_sem=send_sems.at[j],
                recv_sem=recv_sems.at[j],
                device_id=(partner,),
                device_id_type=pl.DeviceIdType.MESH,
            )
            r.start()
            return r

        def stage(name, src, off, m):
            B(name)[:, :] = src[pl.ds(pl.multiple_of(off, 16), m), :].astype(
                jnp.bfloat16
            )

        def addin(off, m, name, base):
            off = pl.multiple_of(off, 16)
            out_ref[pl.ds(off, m), :] = base[pl.ds(off, m), :] + B(name)[
                :, :
            ].astype(jnp.float32)

        def store(off, m, name):
            out_ref[pl.ds(pl.multiple_of(off, 16), m), :] = B(name)[
                :, :
            ].astype(jnp.float32)

        i = lax.axis_index("i")
        dims = _phases(i)
        x0 = x_ref.at[0]

        barrier_sem = pltpu.get_barrier_semaphore()
        for d in ("x", "y1", "z1", "y2", "z2"):
            pl.semaphore_signal(
                barrier_sem,
                inc=1,
                device_id=(dims[d][0],),
                device_id_type=pl.DeviceIdType.MESH,
            )
        pl.semaphore_wait(barrier_sem, 5)

        P = []
        for h in (0, 1):
            bit = [dims[ORDERS[h][s]][1] for s in range(5)]
            par = [dims[ORDERS[h][s]][0] for s in range(5)]
            off = jnp.int32(HALF * h)
            keep, send = [], []
            for s in range(5):
                keep.append(off + bit[s] * ROWS[s])
                send.append(off + (1 - bit[s]) * ROWS[s])
                off = keep[s]
            sf, sr, kf, kr = [], [], [], []
            for s in range(4):
                m, bp = ROWS[s + 1], bit[s + 1]
                sf.append(send[s] + (1 - bp) * m)
                sr.append(send[s] + bp * m)
                kf.append(keep[s] + (1 - bp) * m)
                kr.append(keep[s] + bp * m)
            o, po = [None] * 5, [None] * 5
            o[4] = keep[4]
            for k in range(4, -1, -1):
                po[k] = o[k] + (1 - 2 * bit[k]) * ROWS[k]
                if k:
                    o[k - 1] = o[k] - bit[k] * ROWS[k]
            P.append(
                dict(bit=bit, par=par, keep=keep, send=send, sf=sf, sr=sr,
                     kf=kf, kr=kr, o=o, po=po)
            )

        rs_f = [[None] * 4, [None] * 4]
        rs_r = [[None] * 4, [None] * 4]
        m4a = [None, None]
        m4b = [None, None]
        for h in (0, 1):
            stage(f"rs_sf{h}0", x0, P[h]["sf"][0], ROWS[1])
            rs_f[h][0] = rdma(f"rs_sf{h}0", f"rs_rf{h}0", P[h]["par"][0])
        for h in (0, 1):
            stage(f"rs_sr{h}0", x0, P[h]["sr"][0], ROWS[1])
            rs_r[h][0] = rdma(f"rs_sr{h}0", f"rs_rr{h}0", P[h]["par"][0])
        for s in range(4):
            m = ROWS[s + 1]
            for h in (0, 1):
                rs_f[h][s].wait()
                addin(P[h]["kf"][s], m, f"rs_rf{h}{s}",
                      x0 if s == 0 else out_ref)
            for h in (0, 1):
                if s < 3:
                    m2 = ROWS[s + 2]
                    stage(f"rs_sf{h}{s + 1}", out_ref, P[h]["sf"][s + 1], m2)
                    rs_f[h][s + 1] = rdma(
                        f"rs_sf{h}{s + 1}", f"rs_rf{h}{s + 1}",
                        P[h]["par"][s + 1],
                    )
                    stage(f"rs_sr{h}{s + 1}", out_ref, P[h]["sr"][s + 1], m2)
                    rs_r[h][s + 1] = rdma(
                        f"rs_sr{h}{s + 1}", f"rs_rr{h}{s + 1}",
                        P[h]["par"][s + 1],
                    )
                else:
                    stage(f"m4_sa_{h}", out_ref, P[h]["kf"][3], 16)
                    m4a[h] = rdma(f"m4_sa_{h}", f"m4_ra_{h}", P[h]["par"][4])
            for h in (0, 1):
                rs_r[h][s].wait()
                addin(P[h]["kr"][s], m, f"rs_rr{h}{s}",
                      x0 if s == 0 else out_ref)
            if s == 3:
                for h in (0, 1):
                    stage(f"m4_sb_{h}", out_ref, P[h]["kr"][3], 16)
                    m4b[h] = rdma(f"m4_sb_{h}", f"m4_rb_{h}", P[h]["par"][4])

        ag3a = [None, None]
        ag3b = [None, None]
        agO = [[None] * 3, [None] * 3]
        agN = [[None] * 3, [None] * 3]
        for h in (0, 1):
            m4a[h].wait()
            addin(P[h]["kr"][3], 16, f"m4_ra_{h}", out_ref)
        for h in (0, 1):
            stage(f"ag3_sa_{h}", out_ref, P[h]["kr"][3], 16)
            ag3a[h] = rdma(f"ag3_sa_{h}", f"ag3_ra_{h}", P[h]["par"][3])
        for h in (0, 1):
            m4b[h].wait()
            addin(P[h]["kf"][3], 16, f"m4_rb_{h}", out_ref)
        for h in (0, 1):
            stage(f"ag3_sb_{h}", out_ref, P[h]["kf"][3], 16)
            ag3b[h] = rdma(f"ag3_sb_{h}", f"ag3_rb_{h}", P[h]["par"][3])
            stage(f"ag_so{h}2", out_ref, P[h]["o"][3], 32)
            agO[h][2] = rdma(f"ag_so{h}2", f"ag_ro{h}2", P[h]["par"][2])

        for h in (0, 1):
            b4 = P[h]["bit"][4]
            ag3a[h].wait()
            store(P[h]["po"][3] + b4 * 16, 16, f"ag3_ra_{h}")
            ag3b[h].wait()
            store(P[h]["po"][3] + (1 - b4) * 16, 16, f"ag3_rb_{h}")
        for h in (0, 1):
            stage(f"ag_sn{h}2", out_ref, P[h]["po"][3], 32)
            agN[h][2] = rdma(f"ag_sn{h}2", f"ag_rn{h}2", P[h]["par"][2])
            stage(f"ag_so{h}1", out_ref, P[h]["o"][2], 64)
            agO[h][1] = rdma(f"ag_so{h}1", f"ag_ro{h}1", P[h]["par"][1])
        for k in (1, 0):
            m = ROWS[k + 1]
            m2 = ROWS[k + 2]
            for h in (0, 1):
                b = P[h]["bit"][k + 2]
                agO[h][k + 1].wait()
                store(P[h]["po"][k + 1] + b * m2, m2, f"ag_ro{h}{k + 1}")
                agN[h][k + 1].wait()
                store(P[h]["po"][k + 1] + (1 - b) * m2, m2,
                      f"ag_rn{h}{k + 1}")
            for h in (0, 1):
                stage(f"ag_sn{h}{k}", out_ref, P[h]["po"][k + 1], m)
                agN[h][k] = rdma(f"ag_sn{h}{k}", f"ag_rn{h}{k}",
                                 P[h]["par"][k])
                if k >= 1:
                    stage(f"ag_so{h}{k - 1}", out_ref, P[h]["o"][k], ROWS[k])
                    agO[h][k - 1] = rdma(
                        f"ag_so{h}{k - 1}", f"ag_ro{h}{k - 1}",
                        P[h]["par"][k - 1],
                    )
        for h in (0, 1):
            b = P[h]["bit"][1]
            m2 = ROWS[1]
            agO[h][0].wait()
            store(P[h]["po"][0] + b * m2, m2, f"ag_ro{h}0")
            agN[h][0].wait()
            store(P[h]["po"][0] + (1 - b) * m2, m2, f"ag_rn{h}0")

    return pl.pallas_call(
        body,
        out_shape=jax.ShapeDtypeStruct((M, N), jnp.float32),
        in_specs=[pl.BlockSpec(memory_space=pltpu.VMEM)],
        out_specs=pl.BlockSpec(memory_space=pltpu.VMEM),
        scratch_shapes=(
            [pltpu.VMEM((r, N), jnp.bfloat16) for r in _SHAPES]
            + [
                pltpu.SemaphoreType.DMA((NSEM,)),
                pltpu.SemaphoreType.DMA((NSEM,)),
            ]
        ),
        compiler_params=pltpu.CompilerParams(collective_id=0),
    )(x)


# device time: 40484 ns/iter; 1.9647x vs baseline; 1.0432x over previous
import jax
import jax.numpy as jnp
from jax import lax
from jax.experimental import pallas as pl
from jax.experimental.pallas import tpu as pltpu

M = 1024
N = 1024
HALF = 512
ROWS = (256, 128, 64, 32, 16)
ORDERS = (("x", "y1", "z1", "y2", "z2"), ("y1", "z1", "x", "z2", "y2"))
SPLIT_RS = (True, True, False, False)

_SHAPES: list[int] = []
_IDX: dict[str, int] = {}


def _buf(name: str, rows: int) -> None:
    _IDX[name] = len(_SHAPES)
    _SHAPES.append(rows)


for _h in (0, 1):
    for _s in range(4):
        if SPLIT_RS[_s]:
            _m2 = ROWS[_s + 2]
            for _tag in ("rs_sf1", "rs_sf2", "rs_rf1", "rs_rf2"):
                _buf(f"{_tag}{_h}{_s}", _m2)
        else:
            for _tag in ("rs_sf", "rs_rf"):
                _buf(f"{_tag}{_h}{_s}", ROWS[_s + 1])
        for _tag in ("rs_sr", "rs_rr"):
            _buf(f"{_tag}{_h}{_s}", ROWS[_s + 1])
    for _tag in ("m4_sa", "m4_sb", "m4_ra", "m4_rb",
                 "ag3_sa", "ag3_sb", "ag3_ra", "ag3_rb"):
        _buf(f"{_tag}_{_h}", 16)
    for _k in range(3):
        _buf(f"ag_so{_h}{_k}", ROWS[_k + 1])
        _buf(f"ag_ro{_h}{_k}", ROWS[_k + 1])
        for _tag in ("ag_sna", "ag_snb", "ag_rna", "ag_rnb"):
            _buf(f"{_tag}{_h}{_k}", ROWS[_k + 2])

NSEM = 46


def _phases(i):
    z = i // 8
    p = i % 8
    y = p // 2
    x = (p + y) % 2

    def logical(xx, yy, zz):
        return zz * 8 + 2 * yy + (xx + yy) % 2

    return {
        "x": (logical(1 - x, y, z), x),
        "y1": (logical(x, y ^ 1, z), y & 1),
        "z1": (logical(x, y, z ^ 1), z & 1),
        "y2": (logical(x, y ^ 2, z), (y >> 1) & 1),
        "z2": (logical(x, y, z ^ 2), (z >> 1) & 1),
    }


def kernel(x):
    def body(x_ref, out_ref, *scratch):
        bufs, send_sems, recv_sems = scratch[:-2], scratch[-2], scratch[-1]

        def B(name):
            return bufs[_IDX[name]]

        sem_ctr = [0]

        def rdma(src_name, dst_name, partner):
            j = sem_ctr[0]
            sem_ctr[0] += 1
            r = pltpu.make_async_remote_copy(
                src_ref=B(src_name),
                dst_ref=B(dst_name),
                send_sem=send_sems.at[j],
                recv_sem=recv_sems.at[j],
                device_id=(partner,),
                device_id_type=pl.DeviceIdType.MESH,
            )
            r.start()
            return r

        def stage(name, src, off, m):
            B(name)[:, :] = src[pl.ds(pl.multiple_of(off, 16), m), :].astype(
                jnp.bfloat16
            )

        def addin(off, m, name, base):
            off = pl.multiple_of(off, 16)
            out_ref[pl.ds(off, m), :] = base[pl.ds(off, m), :] + B(name)[
                :, :
            ].astype(jnp.float32)

        def store(off, m, name):
            out_ref[pl.ds(pl.multiple_of(off, 16), m), :] = B(name)[
                :, :
            ].astype(jnp.float32)

        i = lax.axis_index("i")
        dims = _phases(i)
        x0 = x_ref.at[0]

        barrier_sem = pltpu.get_barrier_semaphore()
        for d in ("x", "y1", "z1", "y2", "z2"):
            pl.semaphore_signal(
                barrier_sem,
                inc=1,
                device_id=(dims[d][0],),
                device_id_type=pl.DeviceIdType.MESH,
            )
        pl.semaphore_wait(barrier_sem, 5)

        P = []
        for h in (0, 1):
            bit = [dims[ORDERS[h][s]][1] for s in range(5)]
            par = [dims[ORDERS[h][s]][0] for s in range(5)]
            off = jnp.int32(HALF * h)
            keep, send = [], []
            for s in range(5):
                keep.append(off + bit[s] * ROWS[s])
                send.append(off + (1 - bit[s]) * ROWS[s])
                off = keep[s]
            sf, sr, kf, kr = [], [], [], []
            for s in range(4):
                m, bp = ROWS[s + 1], bit[s + 1]
                sf.append(send[s] + (1 - bp) * m)
                sr.append(send[s] + bp * m)
                kf.append(keep[s] + (1 - bp) * m)
                kr.append(keep[s] + bp * m)
            o, po = [None] * 5, [None] * 5
            o[4] = keep[4]
            for k in range(4, -1, -1):
                po[k] = o[k] + (1 - 2 * bit[k]) * ROWS[k]
                if k:
                    o[k - 1] = o[k] - bit[k] * ROWS[k]
            P.append(
                dict(bit=bit, par=par, keep=keep, send=send, sf=sf, sr=sr,
                     kf=kf, kr=kr, o=o, po=po)
            )

        rs_f1 = [[None] * 4, [None] * 4]
        rs_f2 = [[None] * 4, [None] * 4]
        rs_r = [[None] * 4, [None] * 4]
        m4a = [None, None]
        m4b = [None, None]

        def issue_f(h, s, src):
            if SPLIT_RS[s]:
                m2, b2 = ROWS[s + 2], P[h]["bit"][s + 2]
                stage(f"rs_sf1{h}{s}", src, P[h]["sf"][s] + (1 - b2) * m2, m2)
                rs_f1[h][s] = rdma(f"rs_sf1{h}{s}", f"rs_rf1{h}{s}",
                                   P[h]["par"][s])
                stage(f"rs_sf2{h}{s}", src, P[h]["sf"][s] + b2 * m2, m2)
                rs_f2[h][s] = rdma(f"rs_sf2{h}{s}", f"rs_rf2{h}{s}",
                                   P[h]["par"][s])
            else:
                stage(f"rs_sf{h}{s}", src, P[h]["sf"][s], ROWS[s + 1])
                rs_f1[h][s] = rdma(f"rs_sf{h}{s}", f"rs_rf{h}{s}",
                                   P[h]["par"][s])

        def issue_r(h, s, src):
            stage(f"rs_sr{h}{s}", src, P[h]["sr"][s], ROWS[s + 1])
            rs_r[h][s] = rdma(f"rs_sr{h}{s}", f"rs_rr{h}{s}", P[h]["par"][s])

        for h in (0, 1):
            issue_f(h, 0, x0)
        for h in (0, 1):
            issue_r(h, 0, x0)
        for s in range(4):
            base = x0 if s == 0 else out_ref
            for h in (0, 1):
                rs_f1[h][s].wait()
                if SPLIT_RS[s]:
                    m2, b2 = ROWS[s + 2], P[h]["bit"][s + 2]
                    addin(P[h]["kf"][s] + (1 - b2) * m2, m2,
                          f"rs_rf1{h}{s}", base)
                else:
                    addin(P[h]["kf"][s], ROWS[s + 1], f"rs_rf{h}{s}", base)
            for h in (0, 1):
                if s < 3:
                    issue_f(h, s + 1, out_ref)
                else:
                    stage(f"m4_sa_{h}", out_ref, P[h]["kf"][3], 16)
                    m4a[h] = rdma(f"m4_sa_{h}", f"m4_ra_{h}", P[h]["par"][4])
            if SPLIT_RS[s]:
                for h in (0, 1):
                    m2, b2 = ROWS[s + 2], P[h]["bit"][s + 2]
                    rs_f2[h][s].wait()
                    addin(P[h]["kf"][s] + b2 * m2, m2, f"rs_rf2{h}{s}", base)
                for h in (0, 1):
                    issue_r(h, s + 1, out_ref)
            for h in (0, 1):
                rs_r[h][s].wait()
                addin(P[h]["kr"][s], ROWS[s + 1], f"rs_rr{h}{s}", base)
            if not SPLIT_RS[s] and s < 3:
                for h in (0, 1):
                    issue_r(h, s + 1, out_ref)
            if s == 3:
                for h in (0, 1):
                    stage(f"m4_sb_{h}", out_ref, P[h]["kr"][3], 16)
                    m4b[h] = rdma(f"m4_sb_{h}", f"m4_rb_{h}", P[h]["par"][4])

        ag3a = [None, None]
        ag3b = [None, None]
        agO = [[None] * 3, [None] * 3]
        agNa = [[None] * 3, [None] * 3]
        agNb = [[None] * 3, [None] * 3]
        for h in (0, 1):
            m4a[h].wait()
            addin(P[h]["kr"][3], 16, f"m4_ra_{h}", out_ref)
        for h in (0, 1):
            stage(f"ag3_sa_{h}", out_ref, P[h]["kr"][3], 16)
            ag3a[h] = rdma(f"ag3_sa_{h}", f"ag3_ra_{h}", P[h]["par"][3])
        for h in (0, 1):
            m4b[h].wait()
            addin(P[h]["kf"][3], 16, f"m4_rb_{h}", out_ref)
        for h in (0, 1):
            stage(f"ag3_sb_{h}", out_ref, P[h]["kf"][3], 16)
            ag3b[h] = rdma(f"ag3_sb_{h}", f"ag3_rb_{h}", P[h]["par"][3])
            stage(f"ag_so{h}2", out_ref, P[h]["o"][3], 32)
            agO[h][2] = rdma(f"ag_so{h}2", f"ag_ro{h}2", P[h]["par"][2])

        for h in (0, 1):
            b4 = P[h]["bit"][4]
            ag3a[h].wait()
            store(P[h]["po"][3] + b4 * 16, 16, f"ag3_ra_{h}")
        for h in (0, 1):
            b4 = P[h]["bit"][4]
            stage(f"ag_sna{h}2", out_ref, P[h]["po"][3] + b4 * 16, 16)
            agNa[h][2] = rdma(f"ag_sna{h}2", f"ag_rna{h}2", P[h]["par"][2])
        for h in (0, 1):
            b4 = P[h]["bit"][4]
            ag3b[h].wait()
            store(P[h]["po"][3] + (1 - b4) * 16, 16, f"ag3_rb_{h}")
        for h in (0, 1):
            b4 = P[h]["bit"][4]
            stage(f"ag_snb{h}2", out_ref, P[h]["po"][3] + (1 - b4) * 16, 16)
            agNb[h][2] = rdma(f"ag_snb{h}2", f"ag_rnb{h}2", P[h]["par"][2])
            stage(f"ag_so{h}1", out_ref, P[h]["o"][2], 64)
            agO[h][1] = rdma(f"ag_so{h}1", f"ag_ro{h}1", P[h]["par"][1])

        for k in (1, 0):
            m2 = ROWS[k + 2]
            m3 = ROWS[k + 3]
            for h in (0, 1):
                b = P[h]["bit"][k + 2]
                agO[h][k + 1].wait()
                store(P[h]["po"][k + 1] + b * m2, m2, f"ag_ro{h}{k + 1}")
            for h in (0, 1):
                b = P[h]["bit"][k + 2]
                stage(f"ag_sna{h}{k}", out_ref, P[h]["po"][k + 1] + b * m2,
                      m2)
                agNa[h][k] = rdma(f"ag_sna{h}{k}", f"ag_rna{h}{k}",
                                  P[h]["par"][k])
            for h in (0, 1):
                b = P[h]["bit"][k + 2]
                b3 = P[h]["bit"][k + 3]
                nb = P[h]["po"][k + 1] + (1 - b) * m2
                agNa[h][k + 1].wait()
                store(nb + b3 * m3, m3, f"ag_rna{h}{k + 1}")
                agNb[h][k + 1].wait()
                store(nb + (1 - b3) * m3, m3, f"ag_rnb{h}{k + 1}")
            for h in (0, 1):
                b = P[h]["bit"][k + 2]
                stage(f"ag_snb{h}{k}", out_ref,
                      P[h]["po"][k + 1] + (1 - b) * m2, m2)
                agNb[h][k] = rdma(f"ag_snb{h}{k}", f"ag_rnb{h}{k}",
                                  P[h]["par"][k])
                if k >= 1:
                    stage(f"ag_so{h}{k - 1}", out_ref, P[h]["o"][k], ROWS[k])
                    agO[h][k - 1] = rdma(
                        f"ag_so{h}{k - 1}", f"ag_ro{h}{k - 1}",
                        P[h]["par"][k - 1],
                    )
        for h in (0, 1):
            b1 = P[h]["bit"][1]
            b2 = P[h]["bit"][2]
            agO[h][0].wait()
            store(P[h]["po"][0] + b1 * ROWS[1], ROWS[1], f"ag_ro{h}0")
            nb = P[h]["po"][0] + (1 - b1) * ROWS[1]
            agNa[h][0].wait()
            store(nb + b2 * ROWS[2], ROWS[2], f"ag_rna{h}0")
            agNb[h][0].wait()
            store(nb + (1 - b2) * ROWS[2], ROWS[2], f"ag_rnb{h}0")

    return pl.pallas_call(
        body,
        out_shape=jax.ShapeDtypeStruct((M, N), jnp.float32),
        in_specs=[pl.BlockSpec(memory_space=pltpu.VMEM)],
        out_specs=pl.BlockSpec(memory_space=pltpu.VMEM),
        scratch_shapes=(
            [pltpu.VMEM((r, N), jnp.bfloat16) for r in _SHAPES]
            + [
                pltpu.SemaphoreType.DMA((NSEM,)),
                pltpu.SemaphoreType.DMA((NSEM,)),
            ]
        ),
        compiler_params=pltpu.CompilerParams(collective_id=0),
    )(x)


# device time: 39708 ns/iter; 2.0031x vs baseline; 1.0195x over previous
import jax
import jax.numpy as jnp
from jax import lax
from jax.experimental import pallas as pl
from jax.experimental.pallas import tpu as pltpu

M = 1024
N = 1024
HALF = 512
ROWS = (256, 128, 64, 32, 16)
ORDERS = (("x", "y1", "z1", "y2", "z2"), ("y1", "z1", "x", "z2", "y2"))
SPLIT_RS = (True, True, False, False)

_SHAPES: list[int] = []
_IDX: dict[str, int] = {}


def _buf(name: str, rows: int) -> None:
    _IDX[name] = len(_SHAPES)
    _SHAPES.append(rows)


for _h in (0, 1):
    for _s in range(4):
        if SPLIT_RS[_s]:
            _m2 = ROWS[_s + 2]
            for _tag in ("rs_sf1", "rs_sf2", "rs_rf1", "rs_rf2"):
                _buf(f"{_tag}{_h}{_s}", _m2)
        else:
            for _tag in ("rs_sf", "rs_rf"):
                _buf(f"{_tag}{_h}{_s}", ROWS[_s + 1])
        for _tag in ("rs_sr", "rs_rr"):
            _buf(f"{_tag}{_h}{_s}", ROWS[_s + 1])
    for _tag in ("m4_sa", "m4_sb", "m4_ra", "m4_rb",
                 "ag3_sa", "ag3_sb", "ag3_ra", "ag3_rb"):
        _buf(f"{_tag}_{_h}", 16)
    for _k in range(3):
        _buf(f"ag_so{_h}{_k}", ROWS[_k + 1])
        _buf(f"ag_ro{_h}{_k}", ROWS[_k + 1])
        for _tag in ("ag_sna", "ag_snb", "ag_rna", "ag_rnb"):
            _buf(f"{_tag}{_h}{_k}", ROWS[_k + 2])

NSEM = 46


def _phases(i):
    z = i // 8
    p = i % 8
    y = p // 2
    x = (p + y) % 2

    def logical(xx, yy, zz):
        return zz * 8 + 2 * yy + (xx + yy) % 2

    return {
        "x": (logical(1 - x, y, z), x),
        "y1": (logical(x, y ^ 1, z), y & 1),
        "z1": (logical(x, y, z ^ 1), z & 1),
        "y2": (logical(x, y ^ 2, z), (y >> 1) & 1),
        "z2": (logical(x, y, z ^ 2), (z >> 1) & 1),
    }


def kernel(x):
    def body(x_ref, out_ref, *scratch):
        bufs, send_sems, recv_sems = scratch[:-2], scratch[-2], scratch[-1]

        def B(name):
            return bufs[_IDX[name]]

        sem_ctr = [0]

        def rdma(src_name, dst_name, partner):
            j = sem_ctr[0]
            sem_ctr[0] += 1
            r = pltpu.make_async_remote_copy(
                src_ref=B(src_name),
                dst_ref=B(dst_name),
                send_sem=send_sems.at[j],
                recv_sem=recv_sems.at[j],
                device_id=(partner,),
                device_id_type=pl.DeviceIdType.MESH,
            )
            r.start()
            return r

        def stage(name, src, off, m):
            B(name)[:, :] = src[pl.ds(pl.multiple_of(off, 16), m), :].astype(
                jnp.bfloat16
            )

        def addin(off, m, name, base):
            off = pl.multiple_of(off, 16)
            out_ref[pl.ds(off, m), :] = base[pl.ds(off, m), :] + B(name)[
                :, :
            ].astype(jnp.float32)

        def store(off, m, name):
            out_ref[pl.ds(pl.multiple_of(off, 16), m), :] = B(name)[
                :, :
            ].astype(jnp.float32)

        i = lax.axis_index("i")
        dims = _phases(i)
        x0 = x_ref.at[0]

        barrier_sem = pltpu.get_barrier_semaphore()
        for d in ("x", "y1", "z1", "y2", "z2"):
            pl.semaphore_signal(
                barrier_sem,
                inc=1,
                device_id=(dims[d][0],),
                device_id_type=pl.DeviceIdType.MESH,
            )
        pl.semaphore_wait(barrier_sem, 5)

        P = []
        for h in (0, 1):
            bit = [dims[ORDERS[h][s]][1] for s in range(5)]
            par = [dims[ORDERS[h][s]][0] for s in range(5)]
            off = jnp.int32(HALF * h)
            keep, send = [], []
            for s in range(5):
                keep.append(off + bit[s] * ROWS[s])
                send.append(off + (1 - bit[s]) * ROWS[s])
                off = keep[s]
            sf, sr, kf, kr = [], [], [], []
            for s in range(4):
                m, bp = ROWS[s + 1], bit[s + 1]
                sf.append(send[s] + (1 - bp) * m)
                sr.append(send[s] + bp * m)
                kf.append(keep[s] + (1 - bp) * m)
                kr.append(keep[s] + bp * m)
            o, po = [None] * 5, [None] * 5
            o[4] = keep[4]
            for k in range(4, -1, -1):
                po[k] = o[k] + (1 - 2 * bit[k]) * ROWS[k]
                if k:
                    o[k - 1] = o[k] - bit[k] * ROWS[k]
            P.append(
                dict(bit=bit, par=par, keep=keep, send=send, sf=sf, sr=sr,
                     kf=kf, kr=kr, o=o, po=po)
            )

        rs_f1 = [[None] * 4, [None] * 4]
        rs_f2 = [[None] * 4, [None] * 4]
        rs_r = [[None] * 4, [None] * 4]
        m4a = [None, None]
        m4b = [None, None]

        def issue_f(h, s, src):
            if SPLIT_RS[s]:
                m2, b2 = ROWS[s + 2], P[h]["bit"][s + 2]
                stage(f"rs_sf1{h}{s}", src, P[h]["sf"][s] + (1 - b2) * m2, m2)
                rs_f1[h][s] = rdma(f"rs_sf1{h}{s}", f"rs_rf1{h}{s}",
                                   P[h]["par"][s])
                stage(f"rs_sf2{h}{s}", src, P[h]["sf"][s] + b2 * m2, m2)
                rs_f2[h][s] = rdma(f"rs_sf2{h}{s}", f"rs_rf2{h}{s}",
                                   P[h]["par"][s])
            else:
                stage(f"rs_sf{h}{s}", src, P[h]["sf"][s], ROWS[s + 1])
                rs_f1[h][s] = rdma(f"rs_sf{h}{s}", f"rs_rf{h}{s}",
                                   P[h]["par"][s])

        def issue_r(h, s, src):
            stage(f"rs_sr{h}{s}", src, P[h]["sr"][s], ROWS[s + 1])
            rs_r[h][s] = rdma(f"rs_sr{h}{s}", f"rs_rr{h}{s}", P[h]["par"][s])

        for h in (0, 1):
            issue_f(h, 0, x0)
        for h in (0, 1):
            issue_r(h, 0, x0)
        for s in range(4):
            base = x0 if s == 0 else out_ref
            for h in (0, 1):
                rs_f1[h][s].wait()
                if SPLIT_RS[s]:
                    m2, b2 = ROWS[s + 2], P[h]["bit"][s + 2]
                    addin(P[h]["kf"][s] + (1 - b2) * m2, m2,
                          f"rs_rf1{h}{s}", base)
                else:
                    addin(P[h]["kf"][s], ROWS[s + 1], f"rs_rf{h}{s}", base)
                if s < 3:
                    issue_f(h, s + 1, out_ref)
                else:
                    stage(f"m4_sa_{h}", out_ref, P[h]["kf"][3], 16)
                    m4a[h] = rdma(f"m4_sa_{h}", f"m4_ra_{h}", P[h]["par"][4])
            if SPLIT_RS[s]:
                for h in (0, 1):
                    m2, b2 = ROWS[s + 2], P[h]["bit"][s + 2]
                    rs_f2[h][s].wait()
                    addin(P[h]["kf"][s] + b2 * m2, m2, f"rs_rf2{h}{s}", base)
                    issue_r(h, s + 1, out_ref)
            for h in (0, 1):
                rs_r[h][s].wait()
                addin(P[h]["kr"][s], ROWS[s + 1], f"rs_rr{h}{s}", base)
                if not SPLIT_RS[s] and s < 3:
                    issue_r(h, s + 1, out_ref)
                if s == 3:
                    stage(f"m4_sb_{h}", out_ref, P[h]["kr"][3], 16)
                    m4b[h] = rdma(f"m4_sb_{h}", f"m4_rb_{h}", P[h]["par"][4])

        ag3a = [None, None]
        ag3b = [None, None]
        agO = [[None] * 3, [None] * 3]
        agNa = [[None] * 3, [None] * 3]
        agNb = [[None] * 3, [None] * 3]
        for h in (0, 1):
            m4a[h].wait()
            addin(P[h]["kr"][3], 16, f"m4_ra_{h}", out_ref)
            stage(f"ag3_sa_{h}", out_ref, P[h]["kr"][3], 16)
            ag3a[h] = rdma(f"ag3_sa_{h}", f"ag3_ra_{h}", P[h]["par"][3])
        for h in (0, 1):
            m4b[h].wait()
            addin(P[h]["kf"][3], 16, f"m4_rb_{h}", out_ref)
            stage(f"ag3_sb_{h}", out_ref, P[h]["kf"][3], 16)
            ag3b[h] = rdma(f"ag3_sb_{h}", f"ag3_rb_{h}", P[h]["par"][3])
            stage(f"ag_so{h}2", out_ref, P[h]["o"][3], 32)
            agO[h][2] = rdma(f"ag_so{h}2", f"ag_ro{h}2", P[h]["par"][2])

        for h in (0, 1):
            b4 = P[h]["bit"][4]
            ag3a[h].wait()
            store(P[h]["po"][3] + b4 * 16, 16, f"ag3_ra_{h}")
            stage(f"ag_sna{h}2", out_ref, P[h]["po"][3] + b4 * 16, 16)
            agNa[h][2] = rdma(f"ag_sna{h}2", f"ag_rna{h}2", P[h]["par"][2])
        for h in (0, 1):
            b4 = P[h]["bit"][4]
            ag3b[h].wait()
            store(P[h]["po"][3] + (1 - b4) * 16, 16, f"ag3_rb_{h}")
            stage(f"ag_snb{h}2", out_ref, P[h]["po"][3] + (1 - b4) * 16, 16)
            agNb[h][2] = rdma(f"ag_snb{h}2", f"ag_rnb{h}2", P[h]["par"][2])
            stage(f"ag_so{h}1", out_ref, P[h]["o"][2], 64)
            agO[h][1] = rdma(f"ag_so{h}1", f"ag_ro{h}1", P[h]["par"][1])

        for k in (1, 0):
            m2 = ROWS[k + 2]
            m3 = ROWS[k + 3]
            for h in (0, 1):
                b = P[h]["bit"][k + 2]
                agO[h][k + 1].wait()
                store(P[h]["po"][k + 1] + b * m2, m2, f"ag_ro{h}{k + 1}")
                stage(f"ag_sna{h}{k}", out_ref, P[h]["po"][k + 1] + b * m2,
                      m2)
                agNa[h][k] = rdma(f"ag_sna{h}{k}", f"ag_rna{h}{k}",
                                  P[h]["par"][k])
            for h in (0, 1):
                b = P[h]["bit"][k + 2]
                b3 = P[h]["bit"][k + 3]
                nb = P[h]["po"][k + 1] + (1 - b) * m2
                agNa[h][k + 1].wait()
                store(nb + b3 * m3, m3, f"ag_rna{h}{k + 1}")
                agNb[h][k + 1].wait()
                store(nb + (1 - b3) * m3, m3, f"ag_rnb{h}{k + 1}")
                stage(f"ag_snb{h}{k}", out_ref,
                      P[h]["po"][k + 1] + (1 - b) * m2, m2)
                agNb[h][k] = rdma(f"ag_snb{h}{k}", f"ag_rnb{h}{k}",
                                  P[h]["par"][k])
                if k >= 1:
                    stage(f"ag_so{h}{k - 1}", out_ref, P[h]["o"][k], ROWS[k])
                    agO[h][k - 1] = rdma(
                        f"ag_so{h}{k - 1}", f"ag_ro{h}{k - 1}",
                        P[h]["par"][k - 1],
                    )
        for h in (0, 1):
            b1 = P[h]["bit"][1]
            b2 = P[h]["bit"][2]
            agO[h][0].wait()
            store(P[h]["po"][0] + b1 * ROWS[1], ROWS[1], f"ag_ro{h}0")
            nb = P[h]["po"][0] + (1 - b1) * ROWS[1]
            agNa[h][0].wait()
            store(nb + b2 * ROWS[2], ROWS[2], f"ag_rna{h}0")
            agNb[h][0].wait()
            store(nb + (1 - b2) * ROWS[2], ROWS[2], f"ag_rnb{h}0")

    return pl.pallas_call(
        body,
        out_shape=jax.ShapeDtypeStruct((M, N), jnp.float32),
        in_specs=[pl.BlockSpec(memory_space=pltpu.VMEM)],
        out_specs=pl.BlockSpec(memory_space=pltpu.VMEM),
        scratch_shapes=(
            [pltpu.VMEM((r, N), jnp.bfloat16) for r in _SHAPES]
            + [
                pltpu.SemaphoreType.DMA((NSEM,)),
                pltpu.SemaphoreType.DMA((NSEM,)),
            ]
        ),
        compiler_params=pltpu.CompilerParams(collective_id=0),
    )(x)


# device time: 39701 ns/iter; 2.0035x vs baseline; 1.0002x over previous
import jax
import jax.numpy as jnp
from jax import lax
from jax.experimental import pallas as pl
from jax.experimental.pallas import tpu as pltpu

M = 1024
N = 1024
HALF = 512
ROWS = (256, 128, 64, 32, 16)
ORDERS = (("x", "y1", "z1", "y2", "z2"), ("y1", "z1", "x", "z2", "y2"))
SPLIT_RS = (True, True, False, False)

_SHAPES: list[int] = []
_IDX: dict[str, int] = {}


def _buf(name: str, rows: int) -> None:
    _IDX[name] = len(_SHAPES)
    _SHAPES.append(rows)


for _h in (0, 1):
    for _s in range(4):
        if SPLIT_RS[_s]:
            _m2 = ROWS[_s + 2]
            for _tag in ("rs_sf1", "rs_sf2", "rs_rf1", "rs_rf2"):
                _buf(f"{_tag}{_h}{_s}", _m2)
        else:
            for _tag in ("rs_sf", "rs_rf"):
                _buf(f"{_tag}{_h}{_s}", ROWS[_s + 1])
        for _tag in ("rs_sr", "rs_rr"):
            _buf(f"{_tag}{_h}{_s}", ROWS[_s + 1])
    for _tag in ("m4_sa", "m4_sb", "m4_ra", "m4_rb",
                 "ag3_sa", "ag3_sb", "ag3_ra", "ag3_rb"):
        _buf(f"{_tag}_{_h}", 16)
    for _k in range(3):
        _buf(f"ag_so{_h}{_k}", ROWS[_k + 1])
        _buf(f"ag_ro{_h}{_k}", ROWS[_k + 1])
        for _tag in ("ag_rna", "ag_rnb"):
            _buf(f"{_tag}{_h}{_k}", ROWS[_k + 2])
        if _k < 2:
            _buf(f"ag_snb{_h}{_k}", ROWS[_k + 2])

NSEM = 46


def _phases(i):
    z = i // 8
    p = i % 8
    y = p // 2
    x = (p + y) % 2

    def logical(xx, yy, zz):
        return zz * 8 + 2 * yy + (xx + yy) % 2

    return {
        "x": (logical(1 - x, y, z), x),
        "y1": (logical(x, y ^ 1, z), y & 1),
        "z1": (logical(x, y, z ^ 1), z & 1),
        "y2": (logical(x, y ^ 2, z), (y >> 1) & 1),
        "z2": (logical(x, y, z ^ 2), (z >> 1) & 1),
    }


def kernel(x):
    def body(x_ref, out_ref, *scratch):
        bufs, send_sems, recv_sems = scratch[:-2], scratch[-2], scratch[-1]

        def B(name):
            return bufs[_IDX[name]]

        sem_ctr = [0]

        def rdma(src_name, dst_name, partner):
            j = sem_ctr[0]
            sem_ctr[0] += 1
            r = pltpu.make_async_remote_copy(
                src_ref=B(src_name),
                dst_ref=B(dst_name),
                send_sem=send_sems.at[j],
                recv_sem=recv_sems.at[j],
                device_id=(partner,),
                device_id_type=pl.DeviceIdType.MESH,
            )
            r.start()
            return r

        def stage(name, src, off, m):
            B(name)[:, :] = src[pl.ds(pl.multiple_of(off, 16), m), :].astype(
                jnp.bfloat16
            )

        def addin(off, m, name, base):
            off = pl.multiple_of(off, 16)
            out_ref[pl.ds(off, m), :] = base[pl.ds(off, m), :] + B(name)[
                :, :
            ].astype(jnp.float32)

        def store(off, m, name):
            out_ref[pl.ds(pl.multiple_of(off, 16), m), :] = B(name)[
                :, :
            ].astype(jnp.float32)

        def add_stage(off, m, recv_name, base, send_name):
            off = pl.multiple_of(off, 16)
            v = base[pl.ds(off, m), :] + B(recv_name)[:, :].astype(
                jnp.float32
            )
            B(send_name)[:, :] = v.astype(jnp.bfloat16)
            out_ref[pl.ds(off, m), :] = v

        i = lax.axis_index("i")
        dims = _phases(i)
        x0 = x_ref.at[0]

        barrier_sem = pltpu.get_barrier_semaphore()
        for d in ("x", "y1", "z1", "y2", "z2"):
            pl.semaphore_signal(
                barrier_sem,
                inc=1,
                device_id=(dims[d][0],),
                device_id_type=pl.DeviceIdType.MESH,
            )
        pl.semaphore_wait(barrier_sem, 5)

        P = []
        for h in (0, 1):
            bit = [dims[ORDERS[h][s]][1] for s in range(5)]
            par = [dims[ORDERS[h][s]][0] for s in range(5)]
            off = jnp.int32(HALF * h)
            keep, send = [], []
            for s in range(5):
                keep.append(off + bit[s] * ROWS[s])
                send.append(off + (1 - bit[s]) * ROWS[s])
                off = keep[s]
            sf, sr, kf, kr = [], [], [], []
            for s in range(4):
                m, bp = ROWS[s + 1], bit[s + 1]
                sf.append(send[s] + (1 - bp) * m)
                sr.append(send[s] + bp * m)
                kf.append(keep[s] + (1 - bp) * m)
                kr.append(keep[s] + bp * m)
            o, po = [None] * 5, [None] * 5
            o[4] = keep[4]
            for k in range(4, -1, -1):
                po[k] = o[k] + (1 - 2 * bit[k]) * ROWS[k]
                if k:
                    o[k - 1] = o[k] - bit[k] * ROWS[k]
            P.append(
                dict(bit=bit, par=par, keep=keep, send=send, sf=sf, sr=sr,
                     kf=kf, kr=kr, o=o, po=po)
            )

        rs_f1 = [[None] * 4, [None] * 4]
        rs_f2 = [[None] * 4, [None] * 4]
        rs_r = [[None] * 4, [None] * 4]
        m4a = [None, None]
        m4b = [None, None]

        def issue_f(h, s, src):
            if SPLIT_RS[s]:
                m2, b2 = ROWS[s + 2], P[h]["bit"][s + 2]
                stage(f"rs_sf1{h}{s}", src, P[h]["sf"][s] + (1 - b2) * m2, m2)
                rs_f1[h][s] = rdma(f"rs_sf1{h}{s}", f"rs_rf1{h}{s}",
                                   P[h]["par"][s])
                stage(f"rs_sf2{h}{s}", src, P[h]["sf"][s] + b2 * m2, m2)
                rs_f2[h][s] = rdma(f"rs_sf2{h}{s}", f"rs_rf2{h}{s}",
                                   P[h]["par"][s])
            else:
                stage(f"rs_sf{h}{s}", src, P[h]["sf"][s], ROWS[s + 1])
                rs_f1[h][s] = rdma(f"rs_sf{h}{s}", f"rs_rf{h}{s}",
                                   P[h]["par"][s])

        def issue_r(h, s, src):
            stage(f"rs_sr{h}{s}", src, P[h]["sr"][s], ROWS[s + 1])
            rs_r[h][s] = rdma(f"rs_sr{h}{s}", f"rs_rr{h}{s}", P[h]["par"][s])

        for h in (0, 1):
            issue_f(h, 0, x0)
        for h in (0, 1):
            issue_r(h, 0, x0)
        for s in range(4):
            base = x0 if s == 0 else out_ref
            for h in (0, 1):
                rs_f1[h][s].wait()
                if SPLIT_RS[s]:
                    m2, b2 = ROWS[s + 2], P[h]["bit"][s + 2]
                    q1 = P[h]["kf"][s] + (1 - b2) * m2
                    if s == 1:
                        add_stage(q1, m2, f"rs_rf1{h}{s}", base,
                                  f"rs_sf{h}2")
                        rs_f1[h][2] = rdma(f"rs_sf{h}2", f"rs_rf{h}2",
                                           P[h]["par"][2])
                    else:
                        addin(q1, m2, f"rs_rf1{h}{s}", base)
                        issue_f(h, s + 1, out_ref)
                elif s == 3:
                    add_stage(P[h]["kf"][3], 16, f"rs_rf{h}3", base,
                              f"m4_sa_{h}")
                    m4a[h] = rdma(f"m4_sa_{h}", f"m4_ra_{h}", P[h]["par"][4])
                else:
                    addin(P[h]["kf"][s], ROWS[s + 1], f"rs_rf{h}{s}", base)
                    issue_f(h, s + 1, out_ref)
            if SPLIT_RS[s]:
                for h in (0, 1):
                    m2, b2 = ROWS[s + 2], P[h]["bit"][s + 2]
                    add_stage_r = P[h]["kf"][s] + b2 * m2
                    rs_f2[h][s].wait()
                    add_stage(add_stage_r, m2, f"rs_rf2{h}{s}", base,
                              f"rs_sr{h}{s + 1}")
                    rs_r[h][s + 1] = rdma(f"rs_sr{h}{s + 1}",
                                          f"rs_rr{h}{s + 1}",
                                          P[h]["par"][s + 1])
            for h in (0, 1):
                rs_r[h][s].wait()
                if s == 3:
                    add_stage(P[h]["kr"][3], 16, f"rs_rr{h}3", base,
                              f"m4_sb_{h}")
                    m4b[h] = rdma(f"m4_sb_{h}", f"m4_rb_{h}", P[h]["par"][4])
                else:
                    addin(P[h]["kr"][s], ROWS[s + 1], f"rs_rr{h}{s}", base)
                    if not SPLIT_RS[s]:
                        issue_r(h, s + 1, out_ref)

        ag3a = [None, None]
        ag3b = [None, None]
        agO = [[None] * 3, [None] * 3]
        agNa = [[None] * 3, [None] * 3]
        agNb = [[None] * 3, [None] * 3]
        for h in (0, 1):
            m4a[h].wait()
            add_stage(P[h]["kr"][3], 16, f"m4_ra_{h}", out_ref,
                      f"ag3_sa_{h}")
            ag3a[h] = rdma(f"ag3_sa_{h}", f"ag3_ra_{h}", P[h]["par"][3])
        for h in (0, 1):
            m4b[h].wait()
            add_stage(P[h]["kf"][3], 16, f"m4_rb_{h}", out_ref,
                      f"ag3_sb_{h}")
            ag3b[h] = rdma(f"ag3_sb_{h}", f"ag3_rb_{h}", P[h]["par"][3])
            stage(f"ag_so{h}2", out_ref, P[h]["o"][3], 32)
            agO[h][2] = rdma(f"ag_so{h}2", f"ag_ro{h}2", P[h]["par"][2])

        for h in (0, 1):
            b4 = P[h]["bit"][4]
            ag3a[h].wait()
            agNa[h][2] = rdma(f"ag3_ra_{h}", f"ag_rna{h}2", P[h]["par"][2])
            store(P[h]["po"][3] + b4 * 16, 16, f"ag3_ra_{h}")
        for h in (0, 1):
            b4 = P[h]["bit"][4]
            ag3b[h].wait()
            agNb[h][2] = rdma(f"ag3_rb_{h}", f"ag_rnb{h}2", P[h]["par"][2])
            store(P[h]["po"][3] + (1 - b4) * 16, 16, f"ag3_rb_{h}")
            stage(f"ag_so{h}1", out_ref, P[h]["o"][2], 64)
            agO[h][1] = rdma(f"ag_so{h}1", f"ag_ro{h}1", P[h]["par"][1])

        for k in (1, 0):
            m2 = ROWS[k + 2]
            m3 = ROWS[k + 3]
            for h in (0, 1):
                b = P[h]["bit"][k + 2]
                agO[h][k + 1].wait()
                agNa[h][k] = rdma(f"ag_ro{h}{k + 1}", f"ag_rna{h}{k}",
                                  P[h]["par"][k])
                store(P[h]["po"][k + 1] + b * m2, m2, f"ag_ro{h}{k + 1}")
            for h in (0, 1):
                b = P[h]["bit"][k + 2]
                b3 = P[h]["bit"][k + 3]
                nb = P[h]["po"][k + 1] + (1 - b) * m2
                agNa[h][k + 1].wait()
                store(nb + b3 * m3, m3, f"ag_rna{h}{k + 1}")
                agNb[h][k + 1].wait()
                store(nb + (1 - b3) * m3, m3, f"ag_rnb{h}{k + 1}")
                stage(f"ag_snb{h}{k}", out_ref,
                      P[h]["po"][k + 1] + (1 - b) * m2, m2)
                agNb[h][k] = rdma(f"ag_snb{h}{k}", f"ag_rnb{h}{k}",
                                  P[h]["par"][k])
                if k >= 1:
                    stage(f"ag_so{h}{k - 1}", out_ref, P[h]["o"][k], ROWS[k])
                    agO[h][k - 1] = rdma(
                        f"ag_so{h}{k - 1}", f"ag_ro{h}{k - 1}",
                        P[h]["par"][k - 1],
                    )
        for h in (0, 1):
            b1 = P[h]["bit"][1]
            b2 = P[h]["bit"][2]
            agO[h][0].wait()
            store(P[h]["po"][0] + b1 * ROWS[1], ROWS[1], f"ag_ro{h}0")
            nb = P[h]["po"][0] + (1 - b1) * ROWS[1]
            agNa[h][0].wait()
            store(nb + b2 * ROWS[2], ROWS[2], f"ag_rna{h}0")
            agNb[h][0].wait()
            store(nb + (1 - b2) * ROWS[2], ROWS[2], f"ag_rnb{h}0")

    return pl.pallas_call(
        body,
        out_shape=jax.ShapeDtypeStruct((M, N), jnp.float32),
        in_specs=[pl.BlockSpec(memory_space=pltpu.VMEM)],
        out_specs=pl.BlockSpec(memory_space=pltpu.VMEM),
        scratch_shapes=(
            [pltpu.VMEM((r, N), jnp.bfloat16) for r in _SHAPES]
            + [
                pltpu.SemaphoreType.DMA((NSEM,)),
                pltpu.SemaphoreType.DMA((NSEM,)),
            ]
        ),
        compiler_params=pltpu.CompilerParams(collective_id=0),
    )(x)
